# Initial kernel scaffold; baseline (speedup 1.0000x reference)
import jax
import jax.numpy as jnp
from jax.experimental import pallas as pl


def kernel(x, edge_index, batch, W1, b1, p1, W2, b2, p2, W3, b3, p3, lW1, lb1, lW2, lb2, lW3, lb3):
    raise NotImplementedError("write your pallas kernel here")


# speedup vs baseline: 26.2570x; 26.2570x over previous
import functools
from typing import Any

import jax
import jax.numpy as jnp
from jax import lax
from jax.experimental import pallas as pl
from jax.experimental.pallas import tpu as pltpu
from jax.experimental.pallas import tpu_sc as plsc

N = 50000
E = 800000
NP = 51200
RQ = 400
DP = 48
NTILES = 32
EPT = 25216
EP = NTILES * EPT
FPT = EP // 16
ROWS_PER_TILE = NP // 16

_mesh = plsc.VectorSubcoreMesh(core_axis_name="c", subcore_axis_name="s")
_sc_params = pltpu.CompilerParams(needs_layout_passes=False,
                                  use_tc_tiling_on_sc=False)


def _sc_deg_body(src_hbm, dst_hbm, m_hbm, out_hbm, m_v, deg_v, sbuf, dbuf):
    c = lax.axis_index("c")
    s = lax.axis_index("s")
    wid = s * 2 + c
    pltpu.sync_copy(m_hbm, m_v)

    def zero_step(i, _):
        deg_v[pl.ds(i * 16, 16)] = jnp.zeros((16,), jnp.float32)
        return 0

    lax.fori_loop(0, NP // 16, zero_step, 0)

    base = wid * EPT

    def edge_chunk(off, nvec):
        pltpu.sync_copy(src_hbm.at[pl.ds(base + off, nvec * 16)],
                        sbuf.at[pl.ds(0, nvec * 16)])
        pltpu.sync_copy(dst_hbm.at[pl.ds(base + off, nvec * 16)],
                        dbuf.at[pl.ds(0, nvec * 16)])

        def step(j, _):
            sv = sbuf[pl.ds(j * 16, 16)]
            dv = dbuf[pl.ds(j * 16, 16)]
            mv = plsc.load_gather(m_v, [sv])
            plsc.addupdate_scatter(deg_v, [dv], mv)
            return 0

        lax.fori_loop(0, nvec, step, 0)

    def chunk_loop(ci, _):
        edge_chunk(ci * 2048, 128)
        return 0

    lax.fori_loop(0, 12, chunk_loop, 0)
    edge_chunk(12 * 2048, 40)
    pltpu.sync_copy(deg_v, out_hbm.at[wid])


def _sc_deg(srcp, dstp, m_flat):
    kern = pl.kernel(
        _sc_deg_body,
        out_type=jax.ShapeDtypeStruct((NTILES, NP), jnp.float32),
        mesh=_mesh,
        compiler_params=_sc_params,
        scratch_types=[
            pltpu.VMEM((NP,), jnp.float32),
            pltpu.VMEM((NP,), jnp.float32),
            pltpu.VMEM((2048,), jnp.int32),
            pltpu.VMEM((2048,), jnp.int32),
        ],
    )
    return kern(srcp, dstp, m_flat)


def _sc_feat_body(src_hbm, dst_hbm, ta_hbm, tb_hbm, out_hbm,
                  sbuf, dbuf, rows, sem, acc):
    c = lax.axis_index("c")
    s = lax.axis_index("s")

    def zrow(i, _):
        rows[i, pl.ds(0, 16)] = jnp.zeros((16,), jnp.float32)
        rows[i, pl.ds(16, 16)] = jnp.zeros((16,), jnp.float32)
        return 0

    lax.fori_loop(0, 128, zrow, 0)
    row0 = s * ROWS_PER_TILE

    def zacc(i, _):
        pltpu.sync_copy(rows, acc.at[pl.ds(row0 + i * 128, 128)])
        return 0

    lax.fori_loop(0, ROWS_PER_TILE // 128, zacc, 0)
    plsc.subcore_barrier()

    base = s * FPT

    def chunk(gi, _):
        off = base + gi * 128
        pltpu.sync_copy(src_hbm.at[pl.ds(off, 128)], sbuf)
        pltpu.sync_copy(dst_hbm.at[pl.ds(off, 128)], dbuf)

        @pl.when(c == 0)
        def _():
            pltpu.async_copy(ta_hbm.at[sbuf], rows, sem).wait()

        @pl.when(c == 1)
        def _():
            pltpu.async_copy(tb_hbm.at[sbuf], rows, sem).wait()

        pltpu.sync_copy(rows, acc.at[dbuf], add=True)
        return 0

    lax.fori_loop(0, FPT // 128, chunk, 0)
    plsc.subcore_barrier()

    def drain(i, _):
        r = row0 + i * 128
        pltpu.sync_copy(acc.at[pl.ds(r, 128)], rows)
        pltpu.sync_copy(rows, out_hbm.at[c, pl.ds(r, 128)])
        return 0

    lax.fori_loop(0, ROWS_PER_TILE // 128, drain, 0)


def _sc_feat(srcp, dstp, ta, tb):
    kern = pl.kernel(
        _sc_feat_body,
        out_type=jax.ShapeDtypeStruct((2, NP, 32), jnp.float32),
        mesh=_mesh,
        compiler_params=_sc_params,
        scratch_types=[
            pltpu.VMEM((128,), jnp.int32),
            pltpu.VMEM((128,), jnp.int32),
            pltpu.VMEM((128, 32), jnp.float32),
            pltpu.SemaphoreType.DMA,
            pltpu.VMEM_SHARED((NP, 32), jnp.float32),
        ],
    )
    return kern(srcp, dstp, ta, tb)


def _tc_deg_body(degp_ref, dis_ref, invdeg_ref):
    deg = jnp.sum(degp_ref[...], axis=0) + 1.0
    dis_ref[...] = lax.rsqrt(deg)
    invdeg_ref[...] = 1.0 / deg


def _tc_deg(degp):
    nblk = RQ // 8
    return pl.pallas_call(
        _tc_deg_body,
        grid=(nblk,),
        in_specs=[pl.BlockSpec((NTILES, 8, 128), lambda i: (0, i, 0))],
        out_specs=[
            pl.BlockSpec((8, 128), lambda i: (i, 0)),
            pl.BlockSpec((8, 128), lambda i: (i, 0)),
        ],
        out_shape=[
            jax.ShapeDtypeStruct((RQ, 128), jnp.float32),
            jax.ShapeDtypeStruct((RQ, 128), jnp.float32),
        ],
    )(degp.reshape(NTILES, RQ, 128))


def _tc_prep_body(h_ref, dis_ref, m_ref, w_ref, hl_ref, ta_ref, tb_ref):
    hl = jnp.dot(h_ref[...], w_ref[...], preferred_element_type=jnp.float32)
    hl_ref[...] = hl
    hls = hl * (m_ref[...] * dis_ref[...])
    ta_ref[...] = hls[:, :32]
    tb_ref[...] = jnp.concatenate(
        [hls[:, 32:48], jnp.zeros((hls.shape[0], 16), jnp.float32)], axis=1)


def _tc_prep(h, dis_c, m_c, w48):
    nblk = RQ // 8
    return pl.pallas_call(
        _tc_prep_body,
        grid=(nblk,),
        in_specs=[
            pl.BlockSpec((1024, DP), lambda i: (i, 0)),
            pl.BlockSpec((1024, 1), lambda i: (i, 0)),
            pl.BlockSpec((1024, 1), lambda i: (i, 0)),
            pl.BlockSpec((DP, DP), lambda i: (0, 0)),
        ],
        out_specs=[
            pl.BlockSpec((1024, DP), lambda i: (i, 0)),
            pl.BlockSpec((1024, 32), lambda i: (i, 0)),
            pl.BlockSpec((1024, 32), lambda i: (i, 0)),
        ],
        out_shape=[
            jax.ShapeDtypeStruct((NP, DP), jnp.float32),
            jax.ShapeDtypeStruct((NP, 32), jnp.float32),
            jax.ShapeDtypeStruct((NP, 32), jnp.float32),
        ],
    )(h, dis_c, m_c, w48)


def _tc_post_body(agg_ref, dis_ref, invdeg_ref, hl_ref, m_ref, b_ref, p_ref,
                  hr_ref, s_ref):
    agg0 = jnp.concatenate([agg_ref[0], agg_ref[1][:, :16]], axis=1)
    out = (dis_ref[...] * agg0 + hl_ref[...] * invdeg_ref[...] + b_ref[...])
    hr = jnp.maximum(out, 0.0)
    hr_ref[...] = hr
    p = p_ref[...]
    pnorm = lax.rsqrt(jnp.sum(p * p))
    sc = jnp.sum(hr * p, axis=1, keepdims=True) * pnorm
    s_ref[...] = jnp.where(m_ref[...] > 0.0, sc, -jnp.inf)


def _tc_post(agg, dis_c, invdeg_c, hl, m_c, b48, p48):
    nblk = RQ // 8
    return pl.pallas_call(
        _tc_post_body,
        grid=(nblk,),
        in_specs=[
            pl.BlockSpec((2, 1024, 32), lambda i: (0, i, 0)),
            pl.BlockSpec((1024, 1), lambda i: (i, 0)),
            pl.BlockSpec((1024, 1), lambda i: (i, 0)),
            pl.BlockSpec((1024, DP), lambda i: (i, 0)),
            pl.BlockSpec((1024, 1), lambda i: (i, 0)),
            pl.BlockSpec((1, DP), lambda i: (0, 0)),
            pl.BlockSpec((1, DP), lambda i: (0, 0)),
        ],
        out_specs=[
            pl.BlockSpec((1024, DP), lambda i: (i, 0)),
            pl.BlockSpec((1024, 1), lambda i: (i, 0)),
        ],
        out_shape=[
            jax.ShapeDtypeStruct((NP, DP), jnp.float32),
            jax.ShapeDtypeStruct((NP, 1), jnp.float32),
        ],
    )(agg, dis_c, invdeg_c, hl, m_c, b48, p48)


def _key_from_score(s):
    bits = lax.bitcast_convert_type(s, jnp.uint32)
    return jnp.where(s < 0, ~bits, bits | jnp.uint32(0x80000000))


def _tc_thresh_body(k, s_ref, t_ref):
    key = _key_from_score(s_ref[...])

    def step(i, t):
        tp = t | (jnp.uint32(1) << (jnp.uint32(31) - i.astype(jnp.uint32)))
        cnt = jnp.sum((key >= tp).astype(jnp.int32))
        return jnp.where(cnt >= k, tp, t)

    t = lax.fori_loop(0, 32, step, jnp.uint32(0))
    t_ref[...] = jnp.full((1, 1), t, jnp.uint32)


def _tc_thresh(s, k):
    return pl.pallas_call(
        functools.partial(_tc_thresh_body, k),
        grid=(1,),
        in_specs=[pl.BlockSpec((RQ, 128), lambda i: (0, 0))],
        out_specs=pl.BlockSpec((1, 1), lambda i: (0, 0)),
        out_shape=jax.ShapeDtypeStruct((1, 1), jnp.uint32),
    )(s)


def _tc_apply_body(hr_ref, s_ref, t_ref, h_ref, m_ref, gsum_ref):
    s = s_ref[...]
    key = _key_from_score(s)
    mnew = (key >= t_ref[0, 0]).astype(jnp.float32)
    m_ref[...] = mnew
    gate = jnp.tanh(s) * mnew
    hn = hr_ref[...] * gate
    h_ref[...] = hn

    @pl.when(pl.program_id(0) == 0)
    def _():
        gsum_ref[...] = jnp.zeros_like(gsum_ref)

    gsum_ref[...] += jnp.sum(hn, axis=0, keepdims=True)


def _tc_apply(hr, s, t):
    nblk = RQ // 8
    return pl.pallas_call(
        _tc_apply_body,
        grid=(nblk,),
        in_specs=[
            pl.BlockSpec((1024, DP), lambda i: (i, 0)),
            pl.BlockSpec((1024, 1), lambda i: (i, 0)),
            pl.BlockSpec((1, 1), lambda i: (0, 0)),
        ],
        out_specs=[
            pl.BlockSpec((1024, DP), lambda i: (i, 0)),
            pl.BlockSpec((1024, 1), lambda i: (i, 0)),
            pl.BlockSpec((1, DP), lambda i: (0, 0)),
        ],
        out_shape=[
            jax.ShapeDtypeStruct((NP, DP), jnp.float32),
            jax.ShapeDtypeStruct((NP, 1), jnp.float32),
            jax.ShapeDtypeStruct((1, DP), jnp.float32),
        ],
    )(hr, s, t)


def _tc_mlp_body(g1_ref, g2_ref, g3_ref, w1_ref, b1_ref, w2_ref, b2_ref,
                 w3_ref, b3_ref, o_ref):
    g = (g1_ref[...] * (1.0 / 40000.0) + g2_ref[...] * (1.0 / 32000.0)
         + g3_ref[...] * (1.0 / 25600.0))
    g = jnp.maximum(
        jnp.dot(g, w1_ref[...], preferred_element_type=jnp.float32)
        + b1_ref[...], 0.0)
    g = jnp.maximum(
        jnp.dot(g, w2_ref[...], preferred_element_type=jnp.float32)
        + b2_ref[...], 0.0)
    z = jnp.dot(g, w3_ref[...], preferred_element_type=jnp.float32) + b3_ref[...]
    o_ref[...] = 1.0 / (1.0 + jnp.exp(-z))


def _tc_mlp(g1, g2, g3, lw1, lb1, lw2, lb2, lw3, lb3):
    full = lambda shape: pl.BlockSpec(shape, lambda: tuple(0 for _ in shape))
    return pl.pallas_call(
        _tc_mlp_body,
        in_specs=[
            full((1, DP)), full((1, DP)), full((1, DP)),
            full((DP, DP)), full((1, DP)),
            full((DP, 32)), full((1, 32)),
            full((32, 8)), full((1, 8)),
        ],
        out_specs=full((1, 8)),
        out_shape=jax.ShapeDtypeStruct((1, 8), jnp.float32),
    )(g1, g2, g3, lw1, lb1, lw2, lb2, lw3, lb3)


def kernel(x, edge_index, batch, W1, b1, p1, W2, b2, p2, W3, b3, p3,
           lW1, lb1, lW2, lb2, lW3, lb3):
    f32 = jnp.float32
    w1f = W1.reshape(3, 15, 45).sum(axis=1)

    def pad48(w):
        return jnp.pad(w, ((0, DP - w.shape[0]), (0, DP - w.shape[1])))

    w48 = [pad48(w1f), pad48(W2), pad48(W3)]
    b48 = [jnp.pad(b, (0, DP - 45)).reshape(1, DP) for b in (b1, b2, b3)]
    p48 = [jnp.pad(p, (0, DP - 45)).reshape(1, DP) for p in (p1, p2, p3)]

    h = jnp.pad(x, ((0, NP - N), (0, DP - 3)))
    m_c = jnp.pad(jnp.ones((N,), f32), (0, NP - N)).reshape(NP, 1)

    src = edge_index[0]
    dst = edge_index[1]
    npad = EP - E
    padidx = jnp.full((npad,), N, jnp.int32) + (
        jnp.arange(npad, dtype=jnp.int32) % 1024)
    srcp = jnp.concatenate([src, jnp.zeros((npad,), jnp.int32)])
    dstp = jnp.concatenate([dst, padidx])

    gsums = []
    kvals = [40000, 32000, 25600]
    for w, b, p, k in zip(w48, b48, p48, kvals):
        degp = _sc_deg(srcp, dstp, m_c.reshape(NP))
        dis, invdeg = _tc_deg(degp)
        dis_c = dis.reshape(NP, 1)
        invdeg_c = invdeg.reshape(NP, 1)
        hl, ta, tb = _tc_prep(h, dis_c, m_c, w)
        agg = _sc_feat(srcp, dstp, ta, tb)
        hr, s = _tc_post(agg, dis_c, invdeg_c, hl, m_c, b, p)
        t = _tc_thresh(s.reshape(RQ, 128), k)
        h, m_c, gsum = _tc_apply(hr, s, t)
        gsums.append(gsum)

    lw1 = pad48(lW1)
    lb1p = jnp.pad(lb1, (0, DP - 45)).reshape(1, DP)
    lw2 = jnp.pad(lW2, ((0, DP - 45), (0, 32 - 22)))
    lb2p = jnp.pad(lb2, (0, 32 - 22)).reshape(1, 32)
    lw3 = jnp.pad(lW3, ((0, 32 - 22), (0, 8 - 1)))
    lb3p = jnp.pad(lb3, (0, 8 - 1)).reshape(1, 8)
    o = _tc_mlp(gsums[0], gsums[1], gsums[2], lw1, lb1p, lw2, lb2p, lw3, lb3p)
    return o[0, :1]


# speedup vs baseline: 34.2051x; 1.3027x over previous
import functools
from typing import Any

import jax
import jax.numpy as jnp
from jax import lax
from jax.experimental import pallas as pl
from jax.experimental.pallas import tpu as pltpu
from jax.experimental.pallas import tpu_sc as plsc

N = 50000
E = 800000
NP = 51200
RQ = 400
DP = 48
NTILES = 32
EPT = 25600
EP = NTILES * EPT
FPT = EP // 16
GK = 4
ROWS_PER_TILE = NP // 16

_mesh = plsc.VectorSubcoreMesh(core_axis_name="c", subcore_axis_name="s")
_sc_params = pltpu.CompilerParams(needs_layout_passes=False,
                                  use_tc_tiling_on_sc=False)


def _sc_deg_body(src_hbm, dst_hbm, m_hbm, out_hbm, m_v, deg_v, sbuf, dbuf):
    c = lax.axis_index("c")
    s = lax.axis_index("s")
    wid = s * 2 + c
    pltpu.sync_copy(m_hbm, m_v)

    def zero_step(i, _):
        deg_v[pl.ds(i * 16, 16)] = jnp.zeros((16,), jnp.float32)
        return 0

    lax.fori_loop(0, NP // 16, zero_step, 0)

    base = wid * EPT

    def edge_chunk(off, nvec):
        pltpu.sync_copy(src_hbm.at[pl.ds(base + off, nvec * 16)],
                        sbuf.at[pl.ds(0, nvec * 16)])
        pltpu.sync_copy(dst_hbm.at[pl.ds(base + off, nvec * 16)],
                        dbuf.at[pl.ds(0, nvec * 16)])

        def step(j, _):
            sv = sbuf[pl.ds(j * 16, 16)]
            dv = dbuf[pl.ds(j * 16, 16)]
            mv = plsc.load_gather(m_v, [sv])
            plsc.addupdate_scatter(deg_v, [dv], mv)
            return 0

        lax.fori_loop(0, nvec, step, 0)

    def chunk_loop(ci, _):
        edge_chunk(ci * 1600, 100)
        return 0

    lax.fori_loop(0, EPT // 1600, chunk_loop, 0)
    pltpu.sync_copy(deg_v, out_hbm.at[wid])


def _sc_deg(srcp, dstp, m_flat):
    kern = pl.kernel(
        _sc_deg_body,
        out_type=jax.ShapeDtypeStruct((NTILES, NP), jnp.float32),
        mesh=_mesh,
        compiler_params=_sc_params,
        scratch_types=[
            pltpu.VMEM((NP,), jnp.float32),
            pltpu.VMEM((NP,), jnp.float32),
            pltpu.VMEM((1600,), jnp.int32),
            pltpu.VMEM((1600,), jnp.int32),
        ],
    )
    return kern(srcp, dstp, m_flat)


def _sc_feat_body(src_hbm, dst_hbm, ta_hbm, tb_hbm, out_hbm,
                  sbuf, dbuf, rows, sem, acc):
    c = lax.axis_index("c")
    s = lax.axis_index("s")

    def zrow(i, _):
        rows[0, 0, i, pl.ds(0, 16)] = jnp.zeros((16,), jnp.float32)
        rows[0, 0, i, pl.ds(8, 16)] = jnp.zeros((16,), jnp.float32)
        return 0

    lax.fori_loop(0, 128, zrow, 0)
    row0 = s * ROWS_PER_TILE

    def zacc(i, _):
        pltpu.sync_copy(rows.at[0, 0], acc.at[pl.ds(row0 + i * 128, 128)])
        return 0

    lax.fori_loop(0, ROWS_PER_TILE // 128, zacc, 0)
    plsc.subcore_barrier()

    base = s * FPT

    def load_idx(gi, p):
        off = base + gi * (GK * 128)
        for j in range(GK):
            pltpu.sync_copy(src_hbm.at[pl.ds(off + j * 128, 128)],
                            sbuf.at[p, j])
            pltpu.sync_copy(dst_hbm.at[pl.ds(off + j * 128, 128)],
                            dbuf.at[p, j])

    def fire_gathers(p):
        for j in range(GK):
            @pl.when(c == 0)
            def _():
                pltpu.async_copy(ta_hbm.at[sbuf.at[p, j]],
                                 rows.at[p, j], sem.at[p])

            @pl.when(c == 1)
            def _():
                pltpu.async_copy(tb_hbm.at[sbuf.at[p, j]],
                                 rows.at[p, j], sem.at[p])

    def drain_gathers(p):
        for j in range(GK):
            pltpu.make_async_copy(ta_hbm.at[sbuf.at[p, j]],
                                  rows.at[p, j], sem.at[p]).wait()

    def scatter_group(p):
        for j in range(GK):
            pltpu.sync_copy(rows.at[p, j], acc.at[dbuf.at[p, j]], add=True)

    ngrp = FPT // (GK * 128)
    load_idx(0, 0)
    fire_gathers(0)

    def pipe(i, _):
        p = lax.rem(i, 2)
        q = lax.rem(i + 1, 2)

        @pl.when(i + 1 < ngrp)
        def _():
            load_idx(i + 1, q)
            fire_gathers(q)

        drain_gathers(p)
        scatter_group(p)
        return 0

    lax.fori_loop(0, ngrp, pipe, 0)
    plsc.subcore_barrier()

    def drain(i, _):
        r = row0 + i * 128
        pltpu.sync_copy(acc.at[pl.ds(r, 128)], rows.at[0, 0])
        pltpu.sync_copy(rows.at[0, 0], out_hbm.at[c, pl.ds(r, 128)])
        return 0

    lax.fori_loop(0, ROWS_PER_TILE // 128, drain, 0)


def _sc_feat(srcp, dstp, ta, tb):
    kern = pl.kernel(
        _sc_feat_body,
        out_type=jax.ShapeDtypeStruct((2, NP, 24), jnp.float32),
        mesh=_mesh,
        compiler_params=_sc_params,
        scratch_types=[
            pltpu.VMEM((2, GK, 128), jnp.int32),
            pltpu.VMEM((2, GK, 128), jnp.int32),
            pltpu.VMEM((2, GK, 128, 24), jnp.float32),
            pltpu.SemaphoreType.DMA((2,)),
            pltpu.VMEM_SHARED((NP, 24), jnp.float32),
        ],
    )
    return kern(srcp, dstp, ta, tb)


def _tc_deg_body(degp_ref, dis_ref, invdeg_ref):
    deg = jnp.sum(degp_ref[...], axis=0) + 1.0
    dis_ref[...] = lax.rsqrt(deg)
    invdeg_ref[...] = 1.0 / deg


def _tc_deg(degp):
    nblk = RQ // 8
    return pl.pallas_call(
        _tc_deg_body,
        grid=(nblk,),
        in_specs=[pl.BlockSpec((NTILES, 8, 128), lambda i: (0, i, 0))],
        out_specs=[
            pl.BlockSpec((8, 128), lambda i: (i, 0)),
            pl.BlockSpec((8, 128), lambda i: (i, 0)),
        ],
        out_shape=[
            jax.ShapeDtypeStruct((RQ, 128), jnp.float32),
            jax.ShapeDtypeStruct((RQ, 128), jnp.float32),
        ],
    )(degp.reshape(NTILES, RQ, 128))


def _tc_prep_body(h_ref, dis_ref, m_ref, w_ref, hl_ref, ta_ref, tb_ref):
    hl = jnp.dot(h_ref[...], w_ref[...], preferred_element_type=jnp.float32)
    hl_ref[...] = hl
    hls = hl * (m_ref[...] * dis_ref[...])
    ta_ref[...] = hls[:, :24]
    tb_ref[...] = hls[:, 24:48]


def _tc_prep(h, dis_c, m_c, w48):
    nblk = RQ // 8
    return pl.pallas_call(
        _tc_prep_body,
        grid=(nblk,),
        in_specs=[
            pl.BlockSpec((1024, DP), lambda i: (i, 0)),
            pl.BlockSpec((1024, 1), lambda i: (i, 0)),
            pl.BlockSpec((1024, 1), lambda i: (i, 0)),
            pl.BlockSpec((DP, DP), lambda i: (0, 0)),
        ],
        out_specs=[
            pl.BlockSpec((1024, DP), lambda i: (i, 0)),
            pl.BlockSpec((1024, 24), lambda i: (i, 0)),
            pl.BlockSpec((1024, 24), lambda i: (i, 0)),
        ],
        out_shape=[
            jax.ShapeDtypeStruct((NP, DP), jnp.float32),
            jax.ShapeDtypeStruct((NP, 24), jnp.float32),
            jax.ShapeDtypeStruct((NP, 24), jnp.float32),
        ],
    )(h, dis_c, m_c, w48)


def _tc_post_body(agg_ref, dis_ref, invdeg_ref, hl_ref, m_ref, b_ref, p_ref,
                  hr_ref, s_ref):
    agg0 = jnp.concatenate([agg_ref[0], agg_ref[1]], axis=1)
    out = (dis_ref[...] * agg0 + hl_ref[...] * invdeg_ref[...] + b_ref[...])
    hr = jnp.maximum(out, 0.0)
    hr_ref[...] = hr
    p = p_ref[...]
    pnorm = lax.rsqrt(jnp.sum(p * p))
    sc = jnp.sum(hr * p, axis=1, keepdims=True) * pnorm
    s_ref[...] = jnp.where(m_ref[...] > 0.0, sc, -jnp.inf)


def _tc_post(agg, dis_c, invdeg_c, hl, m_c, b48, p48):
    nblk = RQ // 8
    return pl.pallas_call(
        _tc_post_body,
        grid=(nblk,),
        in_specs=[
            pl.BlockSpec((2, 1024, 24), lambda i: (0, i, 0)),
            pl.BlockSpec((1024, 1), lambda i: (i, 0)),
            pl.BlockSpec((1024, 1), lambda i: (i, 0)),
            pl.BlockSpec((1024, DP), lambda i: (i, 0)),
            pl.BlockSpec((1024, 1), lambda i: (i, 0)),
            pl.BlockSpec((1, DP), lambda i: (0, 0)),
            pl.BlockSpec((1, DP), lambda i: (0, 0)),
        ],
        out_specs=[
            pl.BlockSpec((1024, DP), lambda i: (i, 0)),
            pl.BlockSpec((1024, 1), lambda i: (i, 0)),
        ],
        out_shape=[
            jax.ShapeDtypeStruct((NP, DP), jnp.float32),
            jax.ShapeDtypeStruct((NP, 1), jnp.float32),
        ],
    )(agg, dis_c, invdeg_c, hl, m_c, b48, p48)


def _key_from_score(s):
    bits = lax.bitcast_convert_type(s, jnp.uint32)
    return jnp.where(s < 0, ~bits, bits | jnp.uint32(0x80000000))


def _tc_thresh_body(k, s_ref, t_ref):
    key = _key_from_score(s_ref[...])

    def step(i, t):
        tp = t | (jnp.uint32(1) << (jnp.uint32(31) - i.astype(jnp.uint32)))
        cnt = jnp.sum((key >= tp).astype(jnp.int32))
        return jnp.where(cnt >= k, tp, t)

    t = lax.fori_loop(0, 32, step, jnp.uint32(0))
    t_ref[...] = jnp.full((1, 1), t, jnp.uint32)


def _tc_thresh(s, k):
    return pl.pallas_call(
        functools.partial(_tc_thresh_body, k),
        grid=(1,),
        in_specs=[pl.BlockSpec((RQ, 128), lambda i: (0, 0))],
        out_specs=pl.BlockSpec((1, 1), lambda i: (0, 0)),
        out_shape=jax.ShapeDtypeStruct((1, 1), jnp.uint32),
    )(s)


def _tc_apply_body(hr_ref, s_ref, t_ref, h_ref, m_ref, gsum_ref):
    s = s_ref[...]
    key = _key_from_score(s)
    mnew = (key >= t_ref[0, 0]).astype(jnp.float32)
    m_ref[...] = mnew
    gate = jnp.tanh(s) * mnew
    hn = hr_ref[...] * gate
    h_ref[...] = hn

    @pl.when(pl.program_id(0) == 0)
    def _():
        gsum_ref[...] = jnp.zeros_like(gsum_ref)

    gsum_ref[...] += jnp.sum(hn, axis=0, keepdims=True)


def _tc_apply(hr, s, t):
    nblk = RQ // 8
    return pl.pallas_call(
        _tc_apply_body,
        grid=(nblk,),
        in_specs=[
            pl.BlockSpec((1024, DP), lambda i: (i, 0)),
            pl.BlockSpec((1024, 1), lambda i: (i, 0)),
            pl.BlockSpec((1, 1), lambda i: (0, 0)),
        ],
        out_specs=[
            pl.BlockSpec((1024, DP), lambda i: (i, 0)),
            pl.BlockSpec((1024, 1), lambda i: (i, 0)),
            pl.BlockSpec((1, DP), lambda i: (0, 0)),
        ],
        out_shape=[
            jax.ShapeDtypeStruct((NP, DP), jnp.float32),
            jax.ShapeDtypeStruct((NP, 1), jnp.float32),
            jax.ShapeDtypeStruct((1, DP), jnp.float32),
        ],
    )(hr, s, t)


def _tc_mlp_body(g1_ref, g2_ref, g3_ref, w1_ref, b1_ref, w2_ref, b2_ref,
                 w3_ref, b3_ref, o_ref):
    g = (g1_ref[...] * (1.0 / 40000.0) + g2_ref[...] * (1.0 / 32000.0)
         + g3_ref[...] * (1.0 / 25600.0))
    g = jnp.maximum(
        jnp.dot(g, w1_ref[...], preferred_element_type=jnp.float32)
        + b1_ref[...], 0.0)
    g = jnp.maximum(
        jnp.dot(g, w2_ref[...], preferred_element_type=jnp.float32)
        + b2_ref[...], 0.0)
    z = jnp.dot(g, w3_ref[...], preferred_element_type=jnp.float32) + b3_ref[...]
    o_ref[...] = 1.0 / (1.0 + jnp.exp(-z))


def _tc_mlp(g1, g2, g3, lw1, lb1, lw2, lb2, lw3, lb3):
    full = lambda shape: pl.BlockSpec(shape, lambda: tuple(0 for _ in shape))
    return pl.pallas_call(
        _tc_mlp_body,
        in_specs=[
            full((1, DP)), full((1, DP)), full((1, DP)),
            full((DP, DP)), full((1, DP)),
            full((DP, 32)), full((1, 32)),
            full((32, 8)), full((1, 8)),
        ],
        out_specs=full((1, 8)),
        out_shape=jax.ShapeDtypeStruct((1, 8), jnp.float32),
    )(g1, g2, g3, lw1, lb1, lw2, lb2, lw3, lb3)


def kernel(x, edge_index, batch, W1, b1, p1, W2, b2, p2, W3, b3, p3,
           lW1, lb1, lW2, lb2, lW3, lb3):
    f32 = jnp.float32
    w1f = W1.reshape(3, 15, 45).sum(axis=1)

    def pad48(w):
        return jnp.pad(w, ((0, DP - w.shape[0]), (0, DP - w.shape[1])))

    w48 = [pad48(w1f), pad48(W2), pad48(W3)]
    b48 = [jnp.pad(b, (0, DP - 45)).reshape(1, DP) for b in (b1, b2, b3)]
    p48 = [jnp.pad(p, (0, DP - 45)).reshape(1, DP) for p in (p1, p2, p3)]

    h = jnp.pad(x, ((0, NP - N), (0, DP - 3)))
    m_c = jnp.pad(jnp.ones((N,), f32), (0, NP - N)).reshape(NP, 1)

    src = edge_index[0]
    dst = edge_index[1]
    npad = EP - E
    padidx = jnp.full((npad,), N, jnp.int32) + (
        jnp.arange(npad, dtype=jnp.int32) % 1024)
    srcp = jnp.concatenate([src, jnp.zeros((npad,), jnp.int32)])
    dstp = jnp.concatenate([dst, padidx])

    gsums = []
    kvals = [40000, 32000, 25600]
    for w, b, p, k in zip(w48, b48, p48, kvals):
        degp = _sc_deg(srcp, dstp, m_c.reshape(NP))
        dis, invdeg = _tc_deg(degp)
        dis_c = dis.reshape(NP, 1)
        invdeg_c = invdeg.reshape(NP, 1)
        hl, ta, tb = _tc_prep(h, dis_c, m_c, w)
        agg = _sc_feat(srcp, dstp, ta, tb)
        hr, s = _tc_post(agg, dis_c, invdeg_c, hl, m_c, b, p)
        t = _tc_thresh(s.reshape(RQ, 128), k)
        h, m_c, gsum = _tc_apply(hr, s, t)
        gsums.append(gsum)

    lw1 = pad48(lW1)
    lb1p = jnp.pad(lb1, (0, DP - 45)).reshape(1, DP)
    lw2 = jnp.pad(lW2, ((0, DP - 45), (0, 32 - 22)))
    lb2p = jnp.pad(lb2, (0, 32 - 22)).reshape(1, 32)
    lw3 = jnp.pad(lW3, ((0, 32 - 22), (0, 8 - 1)))
    lb3p = jnp.pad(lb3, (0, 8 - 1)).reshape(1, 8)
    o = _tc_mlp(gsums[0], gsums[1], gsums[2], lw1, lb1p, lw2, lb2p, lw3, lb3p)
    return o[0, :1]


# speedup vs baseline: 44.0092x; 1.2866x over previous
import functools
from typing import Any

import jax
import jax.numpy as jnp
from jax import lax
from jax.experimental import pallas as pl
from jax.experimental.pallas import tpu as pltpu
from jax.experimental.pallas import tpu_sc as plsc

N = 50000
E = 800000
NP = 51200
RQ = 400
DP = 48
NTILES = 32
EPT = 25600
EP = NTILES * EPT
FPT = EP // 16
GK = 4
ROWS_PER_TILE = NP // 16

_mesh = plsc.VectorSubcoreMesh(core_axis_name="c", subcore_axis_name="s")
_sc_params = pltpu.CompilerParams(needs_layout_passes=False,
                                  use_tc_tiling_on_sc=False)


def _sc_deg_body(src_hbm, dst_hbm, m_hbm, out_hbm, m_v, deg_v, sbuf, dbuf):
    c = lax.axis_index("c")
    s = lax.axis_index("s")
    wid = s * 2 + c
    pltpu.sync_copy(m_hbm, m_v)

    def zero_step(i, _):
        deg_v[pl.ds(i * 16, 16)] = jnp.zeros((16,), jnp.float32)
        return 0

    lax.fori_loop(0, NP // 16, zero_step, 0)

    base = wid * EPT

    def edge_chunk(off, nvec):
        pltpu.sync_copy(src_hbm.at[pl.ds(base + off, nvec * 16)],
                        sbuf.at[pl.ds(0, nvec * 16)])
        pltpu.sync_copy(dst_hbm.at[pl.ds(base + off, nvec * 16)],
                        dbuf.at[pl.ds(0, nvec * 16)])

        def step(j, _):
            sv = sbuf[pl.ds(j * 16, 16)]
            dv = dbuf[pl.ds(j * 16, 16)]
            mv = plsc.load_gather(m_v, [sv])
            plsc.addupdate_scatter(deg_v, [dv], mv)
            return 0

        lax.fori_loop(0, nvec, step, 0)

    def chunk_loop(ci, _):
        edge_chunk(ci * 1600, 100)
        return 0

    lax.fori_loop(0, EPT // 1600, chunk_loop, 0)
    pltpu.sync_copy(deg_v, out_hbm.at[wid])


def _sc_deg(srcp, dstp, m_flat):
    kern = pl.kernel(
        _sc_deg_body,
        out_type=jax.ShapeDtypeStruct((NTILES, NP), jnp.float32),
        mesh=_mesh,
        compiler_params=_sc_params,
        scratch_types=[
            pltpu.VMEM((NP,), jnp.float32),
            pltpu.VMEM((NP,), jnp.float32),
            pltpu.VMEM((1600,), jnp.int32),
            pltpu.VMEM((1600,), jnp.int32),
        ],
    )
    return kern(srcp, dstp, m_flat)


IB = 100
NBLK = (FPT // 128) // IB


def _sc_feat_body(src_hbm, dst_hbm, ta_hbm, tb_hbm, out_hbm,
                  sidx, didx, rows, gsem, ssem, acc):
    c = lax.axis_index("c")
    s = lax.axis_index("s")

    def zrow(i, _):
        rows[0, 0, i, pl.ds(0, 16)] = jnp.zeros((16,), jnp.float32)
        rows[0, 0, i, pl.ds(8, 16)] = jnp.zeros((16,), jnp.float32)
        return 0

    lax.fori_loop(0, 128, zrow, 0)
    row0 = s * ROWS_PER_TILE

    def zacc(i, _):
        pltpu.sync_copy(rows.at[0, 0], acc.at[pl.ds(row0 + i * 128, 128)])
        return 0

    lax.fori_loop(0, ROWS_PER_TILE // 128, zacc, 0)
    plsc.subcore_barrier()

    tile_row0 = s * (FPT // 128)
    ngrp = IB // GK

    def fire_gathers(gi, p):
        for j in range(GK):
            @pl.when(c == 0)
            def _():
                pltpu.async_copy(ta_hbm.at[sidx.at[gi * GK + j]],
                                 rows.at[p, j], gsem.at[p])

            @pl.when(c == 1)
            def _():
                pltpu.async_copy(tb_hbm.at[sidx.at[gi * GK + j]],
                                 rows.at[p, j], gsem.at[p])

    def drain_gathers(p):
        for j in range(GK):
            pltpu.make_async_copy(ta_hbm.at[sidx.at[j]],
                                  rows.at[p, j], gsem.at[p]).wait()

    def fire_scatters(gi, p):
        for j in range(GK):
            pltpu.async_copy(rows.at[p, j], acc.at[didx.at[gi * GK + j]],
                             ssem.at[p], add=True)

    def drain_scatters(gi, p):
        for j in range(GK):
            pltpu.make_async_copy(rows.at[p, j],
                                  acc.at[didx.at[gi * GK + j]],
                                  ssem.at[p]).wait()

    for blk in range(NBLK):
        r0 = tile_row0 + blk * IB
        pltpu.sync_copy(src_hbm.at[pl.ds(r0, IB)], sidx)
        pltpu.sync_copy(dst_hbm.at[pl.ds(r0, IB)], didx)
        fire_gathers(0, 0)

        def pipe(i, _):
            p = lax.rem(i, 2)
            q = lax.rem(i + 1, 2)

            @pl.when(i + 1 < ngrp)
            def _():
                @pl.when(i >= 1)
                def _():
                    drain_scatters(i - 1, q)

                fire_gathers(i + 1, q)

            drain_gathers(p)
            fire_scatters(i, p)
            return 0

        lax.fori_loop(0, ngrp, pipe, 0)
        drain_scatters(ngrp - 2, lax.rem(ngrp - 2, 2))
        drain_scatters(ngrp - 1, lax.rem(ngrp - 1, 2))

    plsc.subcore_barrier()

    def drain(i, _):
        r = row0 + i * 128
        pltpu.sync_copy(acc.at[pl.ds(r, 128)], rows.at[0, 0])
        pltpu.sync_copy(rows.at[0, 0], out_hbm.at[c, pl.ds(r, 128)])
        return 0

    lax.fori_loop(0, ROWS_PER_TILE // 128, drain, 0)


def _sc_feat(srcp, dstp, ta, tb):
    kern = pl.kernel(
        _sc_feat_body,
        out_type=jax.ShapeDtypeStruct((2, NP, 24), jnp.float32),
        mesh=_mesh,
        compiler_params=_sc_params,
        scratch_types=[
            pltpu.VMEM((IB, 128), jnp.int32),
            pltpu.VMEM((IB, 128), jnp.int32),
            pltpu.VMEM((2, GK, 128, 24), jnp.float32),
            pltpu.SemaphoreType.DMA((2,)),
            pltpu.SemaphoreType.DMA((2,)),
            pltpu.VMEM_SHARED((NP, 24), jnp.float32),
        ],
    )
    return kern(srcp.reshape(EP // 128, 128), dstp.reshape(EP // 128, 128),
                ta, tb)


def _tc_deg_body(degp_ref, dis_ref, invdeg_ref):
    deg = jnp.sum(degp_ref[...], axis=0) + 1.0
    dis_ref[...] = lax.rsqrt(deg)
    invdeg_ref[...] = 1.0 / deg


def _tc_deg(degp):
    nblk = RQ // 8
    return pl.pallas_call(
        _tc_deg_body,
        grid=(nblk,),
        in_specs=[pl.BlockSpec((NTILES, 8, 128), lambda i: (0, i, 0))],
        out_specs=[
            pl.BlockSpec((8, 128), lambda i: (i, 0)),
            pl.BlockSpec((8, 128), lambda i: (i, 0)),
        ],
        out_shape=[
            jax.ShapeDtypeStruct((RQ, 128), jnp.float32),
            jax.ShapeDtypeStruct((RQ, 128), jnp.float32),
        ],
    )(degp.reshape(NTILES, RQ, 128))


def _tc_prep_body(h_ref, dis_ref, m_ref, w_ref, hl_ref, ta_ref, tb_ref):
    hl = jnp.dot(h_ref[...], w_ref[...], preferred_element_type=jnp.float32)
    hl_ref[...] = hl
    hls = hl * (m_ref[...] * dis_ref[...])
    ta_ref[...] = hls[:, :24]
    tb_ref[...] = hls[:, 24:48]


def _tc_prep(h, dis_c, m_c, w48):
    nblk = RQ // 8
    return pl.pallas_call(
        _tc_prep_body,
        grid=(nblk,),
        in_specs=[
            pl.BlockSpec((1024, DP), lambda i: (i, 0)),
            pl.BlockSpec((1024, 1), lambda i: (i, 0)),
            pl.BlockSpec((1024, 1), lambda i: (i, 0)),
            pl.BlockSpec((DP, DP), lambda i: (0, 0)),
        ],
        out_specs=[
            pl.BlockSpec((1024, DP), lambda i: (i, 0)),
            pl.BlockSpec((1024, 24), lambda i: (i, 0)),
            pl.BlockSpec((1024, 24), lambda i: (i, 0)),
        ],
        out_shape=[
            jax.ShapeDtypeStruct((NP, DP), jnp.float32),
            jax.ShapeDtypeStruct((NP, 24), jnp.float32),
            jax.ShapeDtypeStruct((NP, 24), jnp.float32),
        ],
    )(h, dis_c, m_c, w48)


def _tc_post_body(agg_ref, dis_ref, invdeg_ref, hl_ref, m_ref, b_ref, p_ref,
                  hr_ref, s_ref):
    agg0 = jnp.concatenate([agg_ref[0], agg_ref[1]], axis=1)
    out = (dis_ref[...] * agg0 + hl_ref[...] * invdeg_ref[...] + b_ref[...])
    hr = jnp.maximum(out, 0.0)
    hr_ref[...] = hr
    p = p_ref[...]
    pnorm = lax.rsqrt(jnp.sum(p * p))
    sc = jnp.sum(hr * p, axis=1, keepdims=True) * pnorm
    s_ref[...] = jnp.where(m_ref[...] > 0.0, sc, -jnp.inf)


def _tc_post(agg, dis_c, invdeg_c, hl, m_c, b48, p48):
    nblk = RQ // 8
    return pl.pallas_call(
        _tc_post_body,
        grid=(nblk,),
        in_specs=[
            pl.BlockSpec((2, 1024, 24), lambda i: (0, i, 0)),
            pl.BlockSpec((1024, 1), lambda i: (i, 0)),
            pl.BlockSpec((1024, 1), lambda i: (i, 0)),
            pl.BlockSpec((1024, DP), lambda i: (i, 0)),
            pl.BlockSpec((1024, 1), lambda i: (i, 0)),
            pl.BlockSpec((1, DP), lambda i: (0, 0)),
            pl.BlockSpec((1, DP), lambda i: (0, 0)),
        ],
        out_specs=[
            pl.BlockSpec((1024, DP), lambda i: (i, 0)),
            pl.BlockSpec((1024, 1), lambda i: (i, 0)),
        ],
        out_shape=[
            jax.ShapeDtypeStruct((NP, DP), jnp.float32),
            jax.ShapeDtypeStruct((NP, 1), jnp.float32),
        ],
    )(agg, dis_c, invdeg_c, hl, m_c, b48, p48)


def _key_from_score(s):
    bits = lax.bitcast_convert_type(s, jnp.uint32)
    return jnp.where(s < 0, ~bits, bits | jnp.uint32(0x80000000))


def _tc_thresh_body(k, s_ref, t_ref):
    key = _key_from_score(s_ref[...])

    def step(i, t):
        tp = t | (jnp.uint32(1) << (jnp.uint32(31) - i.astype(jnp.uint32)))
        cnt = jnp.sum((key >= tp).astype(jnp.int32))
        return jnp.where(cnt >= k, tp, t)

    t = lax.fori_loop(0, 32, step, jnp.uint32(0))
    t_ref[...] = jnp.full((1, 1), t, jnp.uint32)


def _tc_thresh(s, k):
    return pl.pallas_call(
        functools.partial(_tc_thresh_body, k),
        grid=(1,),
        in_specs=[pl.BlockSpec((RQ, 128), lambda i: (0, 0))],
        out_specs=pl.BlockSpec((1, 1), lambda i: (0, 0)),
        out_shape=jax.ShapeDtypeStruct((1, 1), jnp.uint32),
    )(s)


def _tc_apply_body(hr_ref, s_ref, t_ref, h_ref, m_ref, gsum_ref):
    s = s_ref[...]
    key = _key_from_score(s)
    mnew = (key >= t_ref[0, 0]).astype(jnp.float32)
    m_ref[...] = mnew
    gate = jnp.tanh(s) * mnew
    hn = hr_ref[...] * gate
    h_ref[...] = hn

    @pl.when(pl.program_id(0) == 0)
    def _():
        gsum_ref[...] = jnp.zeros_like(gsum_ref)

    gsum_ref[...] += jnp.sum(hn, axis=0, keepdims=True)


def _tc_apply(hr, s, t):
    nblk = RQ // 8
    return pl.pallas_call(
        _tc_apply_body,
        grid=(nblk,),
        in_specs=[
            pl.BlockSpec((1024, DP), lambda i: (i, 0)),
            pl.BlockSpec((1024, 1), lambda i: (i, 0)),
            pl.BlockSpec((1, 1), lambda i: (0, 0)),
        ],
        out_specs=[
            pl.BlockSpec((1024, DP), lambda i: (i, 0)),
            pl.BlockSpec((1024, 1), lambda i: (i, 0)),
            pl.BlockSpec((1, DP), lambda i: (0, 0)),
        ],
        out_shape=[
            jax.ShapeDtypeStruct((NP, DP), jnp.float32),
            jax.ShapeDtypeStruct((NP, 1), jnp.float32),
            jax.ShapeDtypeStruct((1, DP), jnp.float32),
        ],
    )(hr, s, t)


def _tc_mlp_body(g1_ref, g2_ref, g3_ref, w1_ref, b1_ref, w2_ref, b2_ref,
                 w3_ref, b3_ref, o_ref):
    g = (g1_ref[...] * (1.0 / 40000.0) + g2_ref[...] * (1.0 / 32000.0)
         + g3_ref[...] * (1.0 / 25600.0))
    g = jnp.maximum(
        jnp.dot(g, w1_ref[...], preferred_element_type=jnp.float32)
        + b1_ref[...], 0.0)
    g = jnp.maximum(
        jnp.dot(g, w2_ref[...], preferred_element_type=jnp.float32)
        + b2_ref[...], 0.0)
    z = jnp.dot(g, w3_ref[...], preferred_element_type=jnp.float32) + b3_ref[...]
    o_ref[...] = 1.0 / (1.0 + jnp.exp(-z))


def _tc_mlp(g1, g2, g3, lw1, lb1, lw2, lb2, lw3, lb3):
    full = lambda shape: pl.BlockSpec(shape, lambda: tuple(0 for _ in shape))
    return pl.pallas_call(
        _tc_mlp_body,
        in_specs=[
            full((1, DP)), full((1, DP)), full((1, DP)),
            full((DP, DP)), full((1, DP)),
            full((DP, 32)), full((1, 32)),
            full((32, 8)), full((1, 8)),
        ],
        out_specs=full((1, 8)),
        out_shape=jax.ShapeDtypeStruct((1, 8), jnp.float32),
    )(g1, g2, g3, lw1, lb1, lw2, lb2, lw3, lb3)


def kernel(x, edge_index, batch, W1, b1, p1, W2, b2, p2, W3, b3, p3,
           lW1, lb1, lW2, lb2, lW3, lb3):
    f32 = jnp.float32
    w1f = W1.reshape(3, 15, 45).sum(axis=1)

    def pad48(w):
        return jnp.pad(w, ((0, DP - w.shape[0]), (0, DP - w.shape[1])))

    w48 = [pad48(w1f), pad48(W2), pad48(W3)]
    b48 = [jnp.pad(b, (0, DP - 45)).reshape(1, DP) for b in (b1, b2, b3)]
    p48 = [jnp.pad(p, (0, DP - 45)).reshape(1, DP) for p in (p1, p2, p3)]

    h = jnp.pad(x, ((0, NP - N), (0, DP - 3)))
    m_c = jnp.pad(jnp.ones((N,), f32), (0, NP - N)).reshape(NP, 1)

    src = edge_index[0]
    dst = edge_index[1]
    npad = EP - E
    padidx = jnp.full((npad,), N, jnp.int32) + (
        jnp.arange(npad, dtype=jnp.int32) % 1024)
    srcp = jnp.concatenate([src, jnp.zeros((npad,), jnp.int32)])
    dstp = jnp.concatenate([dst, padidx])

    gsums = []
    kvals = [40000, 32000, 25600]
    for w, b, p, k in zip(w48, b48, p48, kvals):
        degp = _sc_deg(srcp, dstp, m_c.reshape(NP))
        dis, invdeg = _tc_deg(degp)
        dis_c = dis.reshape(NP, 1)
        invdeg_c = invdeg.reshape(NP, 1)
        hl, ta, tb = _tc_prep(h, dis_c, m_c, w)
        agg = _sc_feat(srcp, dstp, ta, tb)
        hr, s = _tc_post(agg, dis_c, invdeg_c, hl, m_c, b, p)
        t = _tc_thresh(s.reshape(RQ, 128), k)
        h, m_c, gsum = _tc_apply(hr, s, t)
        gsums.append(gsum)

    lw1 = pad48(lW1)
    lb1p = jnp.pad(lb1, (0, DP - 45)).reshape(1, DP)
    lw2 = jnp.pad(lW2, ((0, DP - 45), (0, 32 - 22)))
    lb2p = jnp.pad(lb2, (0, 32 - 22)).reshape(1, 32)
    lw3 = jnp.pad(lW3, ((0, 32 - 22), (0, 8 - 1)))
    lb3p = jnp.pad(lb3, (0, 8 - 1)).reshape(1, 8)
    o = _tc_mlp(gsums[0], gsums[1], gsums[2], lw1, lb1p, lw2, lb2p, lw3, lb3p)
    return o[0, :1]


# speedup vs baseline: 44.2838x; 1.0062x over previous
import functools
from typing import Any

import jax
import jax.numpy as jnp
from jax import lax
from jax.experimental import pallas as pl
from jax.experimental.pallas import tpu as pltpu
from jax.experimental.pallas import tpu_sc as plsc

N = 50000
E = 800000
NP = 51200
RQ = 400
DP = 48
NTILES = 32
EPT = 25600
EP = NTILES * EPT
FPT = EP // 16
GK = 4
ROWS_PER_TILE = NP // 16

_mesh = plsc.VectorSubcoreMesh(core_axis_name="c", subcore_axis_name="s")
_sc_params = pltpu.CompilerParams(needs_layout_passes=False,
                                  use_tc_tiling_on_sc=False)


def _sc_deg_body(src_hbm, dst_hbm, m_hbm, out_hbm, m_v, deg_v, sbuf, dbuf):
    c = lax.axis_index("c")
    s = lax.axis_index("s")
    wid = s * 2 + c
    pltpu.sync_copy(m_hbm, m_v)

    def zero_step(i, _):
        deg_v[pl.ds(i * 16, 16)] = jnp.zeros((16,), jnp.float32)
        return 0

    lax.fori_loop(0, NP // 16, zero_step, 0)

    base = wid * EPT

    def edge_chunk(off, nvec):
        pltpu.sync_copy(src_hbm.at[pl.ds(base + off, nvec * 16)],
                        sbuf.at[pl.ds(0, nvec * 16)])
        pltpu.sync_copy(dst_hbm.at[pl.ds(base + off, nvec * 16)],
                        dbuf.at[pl.ds(0, nvec * 16)])

        def step(j, _):
            sv = sbuf[pl.ds(j * 16, 16)]
            dv = dbuf[pl.ds(j * 16, 16)]
            mv = plsc.load_gather(m_v, [sv])
            plsc.addupdate_scatter(deg_v, [dv], mv)
            return 0

        lax.fori_loop(0, nvec, step, 0)

    def chunk_loop(ci, _):
        edge_chunk(ci * 1600, 100)
        return 0

    lax.fori_loop(0, EPT // 1600, chunk_loop, 0)
    pltpu.sync_copy(deg_v, out_hbm.at[wid])


def _sc_deg(srcp, dstp, m_flat):
    kern = pl.kernel(
        _sc_deg_body,
        out_type=jax.ShapeDtypeStruct((NTILES, NP), jnp.float32),
        mesh=_mesh,
        compiler_params=_sc_params,
        scratch_types=[
            pltpu.VMEM((NP,), jnp.float32),
            pltpu.VMEM((NP,), jnp.float32),
            pltpu.VMEM((1600,), jnp.int32),
            pltpu.VMEM((1600,), jnp.int32),
        ],
    )
    return kern(srcp, dstp, m_flat)


IB = 100
NBLK = (FPT // 128) // IB


def _sc_feat_body(src_hbm, dst_hbm, ta_hbm, tb_hbm, out_hbm,
                  sidx, didx, rows, gsem, ssem, acc):
    c = lax.axis_index("c")
    s = lax.axis_index("s")

    def zrow(i, _):
        rows[0, 0, i, pl.ds(0, 16)] = jnp.zeros((16,), jnp.float32)
        rows[0, 0, i, pl.ds(8, 16)] = jnp.zeros((16,), jnp.float32)
        return 0

    lax.fori_loop(0, 128, zrow, 0)
    row0 = s * ROWS_PER_TILE

    def zacc(i, _):
        pltpu.sync_copy(rows.at[0, 0], acc.at[pl.ds(row0 + i * 128, 128)])
        return 0

    lax.fori_loop(0, ROWS_PER_TILE // 128, zacc, 0)
    plsc.subcore_barrier()

    tile_row0 = s * (FPT // 128)
    ngrp = IB // GK

    def fire_gathers(gi, p):
        for j in range(GK):
            @pl.when(c == 0)
            def _():
                pltpu.async_copy(ta_hbm.at[sidx.at[gi * GK + j]],
                                 rows.at[p, j], gsem.at[p])

            @pl.when(c == 1)
            def _():
                pltpu.async_copy(tb_hbm.at[sidx.at[gi * GK + j]],
                                 rows.at[p, j], gsem.at[p])

    def drain_gathers(p):
        for j in range(GK):
            pltpu.make_async_copy(ta_hbm.at[sidx.at[j]],
                                  rows.at[p, j], gsem.at[p]).wait()

    def fire_scatters(gi, p):
        for j in range(GK):
            pltpu.async_copy(rows.at[p, j], acc.at[didx.at[gi * GK + j]],
                             ssem.at[p], add=True)

    def drain_scatters(gi, p):
        for j in range(GK):
            pltpu.make_async_copy(rows.at[p, j],
                                  acc.at[didx.at[gi * GK + j]],
                                  ssem.at[p]).wait()

    for blk in range(NBLK):
        r0 = tile_row0 + blk * IB
        pltpu.sync_copy(src_hbm.at[pl.ds(r0, IB)], sidx)
        pltpu.sync_copy(dst_hbm.at[pl.ds(r0, IB)], didx)
        fire_gathers(0, 0)

        def pipe(i, _):
            p = lax.rem(i, 2)
            q = lax.rem(i + 1, 2)

            @pl.when(i + 1 < ngrp)
            def _():
                @pl.when(i >= 1)
                def _():
                    drain_scatters(i - 1, q)

                fire_gathers(i + 1, q)

            drain_gathers(p)
            fire_scatters(i, p)
            return 0

        lax.fori_loop(0, ngrp, pipe, 0)
        drain_scatters(ngrp - 2, lax.rem(ngrp - 2, 2))
        drain_scatters(ngrp - 1, lax.rem(ngrp - 1, 2))

    plsc.subcore_barrier()

    def drain(i, _):
        r = row0 + i * 128
        pltpu.sync_copy(acc.at[pl.ds(r, 128)], rows.at[0, 0])
        pltpu.sync_copy(rows.at[0, 0], out_hbm.at[c, pl.ds(r, 128)])
        return 0

    lax.fori_loop(0, ROWS_PER_TILE // 128, drain, 0)


def _sc_feat(srcp, dstp, ta, tb):
    kern = pl.kernel(
        _sc_feat_body,
        out_type=jax.ShapeDtypeStruct((2, NP, 24), jnp.float32),
        mesh=_mesh,
        compiler_params=_sc_params,
        scratch_types=[
            pltpu.VMEM((IB, 128), jnp.int32),
            pltpu.VMEM((IB, 128), jnp.int32),
            pltpu.VMEM((2, GK, 128, 24), jnp.float32),
            pltpu.SemaphoreType.DMA((2,)),
            pltpu.SemaphoreType.DMA((2,)),
            pltpu.VMEM_SHARED((NP, 24), jnp.float32),
        ],
    )
    return kern(srcp.reshape(EP // 128, 128), dstp.reshape(EP // 128, 128),
                ta, tb)


def _tc_deg_body(degp_ref, dis_ref, invdeg_ref):
    deg = jnp.sum(degp_ref[...], axis=0) + 1.0
    dis_ref[...] = lax.rsqrt(deg)
    invdeg_ref[...] = 1.0 / deg


def _tc_deg(degp):
    nblk = RQ // 8
    return pl.pallas_call(
        _tc_deg_body,
        grid=(nblk,),
        in_specs=[pl.BlockSpec((NTILES, 8, 128), lambda i: (0, i, 0))],
        out_specs=[
            pl.BlockSpec((8, 128), lambda i: (i, 0)),
            pl.BlockSpec((8, 128), lambda i: (i, 0)),
        ],
        out_shape=[
            jax.ShapeDtypeStruct((RQ, 128), jnp.float32),
            jax.ShapeDtypeStruct((RQ, 128), jnp.float32),
        ],
    )(degp.reshape(NTILES, RQ, 128))


def _tc_prep_body(h_ref, dis_ref, m_ref, w_ref, hl_ref, ta_ref, tb_ref):
    hl = jnp.dot(h_ref[...], w_ref[...], preferred_element_type=jnp.float32)
    hl_ref[...] = hl
    hls = hl * (m_ref[...] * dis_ref[...])
    ta_ref[...] = hls[:, :24]
    tb_ref[...] = hls[:, 24:48]


def _tc_prep(h, dis_c, m_c, w48):
    nblk = RQ // 8
    return pl.pallas_call(
        _tc_prep_body,
        grid=(nblk,),
        in_specs=[
            pl.BlockSpec((1024, DP), lambda i: (i, 0)),
            pl.BlockSpec((1024, 1), lambda i: (i, 0)),
            pl.BlockSpec((1024, 1), lambda i: (i, 0)),
            pl.BlockSpec((DP, DP), lambda i: (0, 0)),
        ],
        out_specs=[
            pl.BlockSpec((1024, DP), lambda i: (i, 0)),
            pl.BlockSpec((1024, 24), lambda i: (i, 0)),
            pl.BlockSpec((1024, 24), lambda i: (i, 0)),
        ],
        out_shape=[
            jax.ShapeDtypeStruct((NP, DP), jnp.float32),
            jax.ShapeDtypeStruct((NP, 24), jnp.float32),
            jax.ShapeDtypeStruct((NP, 24), jnp.float32),
        ],
    )(h, dis_c, m_c, w48)


def _tc_post_body(agg_ref, dis_ref, invdeg_ref, hl_ref, m_ref, b_ref, p_ref,
                  hr_ref, s_ref):
    agg0 = jnp.concatenate([agg_ref[0], agg_ref[1]], axis=1)
    out = (dis_ref[...] * agg0 + hl_ref[...] * invdeg_ref[...] + b_ref[...])
    hr = jnp.maximum(out, 0.0)
    hr_ref[...] = hr
    p = p_ref[...]
    pnorm = lax.rsqrt(jnp.sum(p * p))
    sc = jnp.sum(hr * p, axis=1, keepdims=True) * pnorm
    s_ref[...] = jnp.where(m_ref[...] > 0.0, sc, -jnp.inf)


def _tc_post(agg, dis_c, invdeg_c, hl, m_c, b48, p48):
    nblk = RQ // 8
    return pl.pallas_call(
        _tc_post_body,
        grid=(nblk,),
        in_specs=[
            pl.BlockSpec((2, 1024, 24), lambda i: (0, i, 0)),
            pl.BlockSpec((1024, 1), lambda i: (i, 0)),
            pl.BlockSpec((1024, 1), lambda i: (i, 0)),
            pl.BlockSpec((1024, DP), lambda i: (i, 0)),
            pl.BlockSpec((1024, 1), lambda i: (i, 0)),
            pl.BlockSpec((1, DP), lambda i: (0, 0)),
            pl.BlockSpec((1, DP), lambda i: (0, 0)),
        ],
        out_specs=[
            pl.BlockSpec((1024, DP), lambda i: (i, 0)),
            pl.BlockSpec((1024, 1), lambda i: (i, 0)),
        ],
        out_shape=[
            jax.ShapeDtypeStruct((NP, DP), jnp.float32),
            jax.ShapeDtypeStruct((NP, 1), jnp.float32),
        ],
    )(agg, dis_c, invdeg_c, hl, m_c, b48, p48)


def _key_from_score(s):
    bits = lax.bitcast_convert_type(s, jnp.uint32)
    return jnp.where(s < 0, ~bits, bits | jnp.uint32(0x80000000))


def _tc_ta_body(k, sp_ref, hr_ref, s_ref, h_ref, m_ref, gsum_ref, t_scr):
    i = pl.program_id(0)

    @pl.when(i == 0)
    def _():
        key = _key_from_score(sp_ref[...])

        def step(b, t):
            tp = t | (jnp.uint32(1) << (jnp.uint32(31) - b.astype(jnp.uint32)))
            cnt = jnp.sum((key >= tp).astype(jnp.int32))
            return jnp.where(cnt >= k, tp, t)

        t_scr[0, 0] = lax.fori_loop(0, 32, step, jnp.uint32(0))

    @pl.when(i > 0)
    def _():
        s = s_ref[...]
        key = _key_from_score(s)
        mnew = (key >= t_scr[0, 0]).astype(jnp.float32)
        m_ref[...] = mnew
        gate = jnp.tanh(s) * mnew
        hn = hr_ref[...] * gate
        h_ref[...] = hn

        @pl.when(i == 1)
        def _():
            gsum_ref[...] = jnp.zeros_like(gsum_ref)

        gsum_ref[...] += jnp.sum(hn, axis=0, keepdims=True)


def _tc_ta(sp, hr, s, k):
    nblk = RQ // 8
    back = lambda i: (jnp.maximum(i - 1, 0), 0)
    return pl.pallas_call(
        functools.partial(_tc_ta_body, k),
        grid=(nblk + 1,),
        in_specs=[
            pl.BlockSpec((RQ, 128), lambda i: (0, 0)),
            pl.BlockSpec((1024, DP), back),
            pl.BlockSpec((1024, 1), back),
        ],
        out_specs=[
            pl.BlockSpec((1024, DP), back),
            pl.BlockSpec((1024, 1), back),
            pl.BlockSpec((1, DP), lambda i: (0, 0)),
        ],
        out_shape=[
            jax.ShapeDtypeStruct((NP, DP), jnp.float32),
            jax.ShapeDtypeStruct((NP, 1), jnp.float32),
            jax.ShapeDtypeStruct((1, DP), jnp.float32),
        ],
        scratch_shapes=[pltpu.SMEM((1, 1), jnp.uint32)],
    )(sp, hr, s)


def _tc_mlp_body(g1_ref, g2_ref, g3_ref, w1_ref, b1_ref, w2_ref, b2_ref,
                 w3_ref, b3_ref, o_ref):
    g = (g1_ref[...] * (1.0 / 40000.0) + g2_ref[...] * (1.0 / 32000.0)
         + g3_ref[...] * (1.0 / 25600.0))
    g = jnp.maximum(
        jnp.dot(g, w1_ref[...], preferred_element_type=jnp.float32)
        + b1_ref[...], 0.0)
    g = jnp.maximum(
        jnp.dot(g, w2_ref[...], preferred_element_type=jnp.float32)
        + b2_ref[...], 0.0)
    z = jnp.dot(g, w3_ref[...], preferred_element_type=jnp.float32) + b3_ref[...]
    o_ref[...] = 1.0 / (1.0 + jnp.exp(-z))


def _tc_mlp(g1, g2, g3, lw1, lb1, lw2, lb2, lw3, lb3):
    full = lambda shape: pl.BlockSpec(shape, lambda: tuple(0 for _ in shape))
    return pl.pallas_call(
        _tc_mlp_body,
        in_specs=[
            full((1, DP)), full((1, DP)), full((1, DP)),
            full((DP, DP)), full((1, DP)),
            full((DP, 32)), full((1, 32)),
            full((32, 8)), full((1, 8)),
        ],
        out_specs=full((1, 8)),
        out_shape=jax.ShapeDtypeStruct((1, 8), jnp.float32),
    )(g1, g2, g3, lw1, lb1, lw2, lb2, lw3, lb3)


def kernel(x, edge_index, batch, W1, b1, p1, W2, b2, p2, W3, b3, p3,
           lW1, lb1, lW2, lb2, lW3, lb3):
    f32 = jnp.float32
    w1f = W1.reshape(3, 15, 45).sum(axis=1)

    def pad48(w):
        return jnp.pad(w, ((0, DP - w.shape[0]), (0, DP - w.shape[1])))

    w48 = [pad48(w1f), pad48(W2), pad48(W3)]
    b48 = [jnp.pad(b, (0, DP - 45)).reshape(1, DP) for b in (b1, b2, b3)]
    p48 = [jnp.pad(p, (0, DP - 45)).reshape(1, DP) for p in (p1, p2, p3)]

    h = jnp.pad(x, ((0, NP - N), (0, DP - 3)))
    m_c = jnp.pad(jnp.ones((N,), f32), (0, NP - N)).reshape(NP, 1)

    src = edge_index[0]
    dst = edge_index[1]
    npad = EP - E
    padidx = jnp.full((npad,), N, jnp.int32) + (
        jnp.arange(npad, dtype=jnp.int32) % 1024)
    srcp = jnp.concatenate([src, jnp.zeros((npad,), jnp.int32)])
    dstp = jnp.concatenate([dst, padidx])

    gsums = []
    kvals = [40000, 32000, 25600]
    for w, b, p, k in zip(w48, b48, p48, kvals):
        degp = _sc_deg(srcp, dstp, m_c.reshape(NP))
        dis, invdeg = _tc_deg(degp)
        dis_c = dis.reshape(NP, 1)
        invdeg_c = invdeg.reshape(NP, 1)
        hl, ta, tb = _tc_prep(h, dis_c, m_c, w)
        agg = _sc_feat(srcp, dstp, ta, tb)
        hr, s = _tc_post(agg, dis_c, invdeg_c, hl, m_c, b, p)
        h, m_c, gsum = _tc_ta(s.reshape(RQ, 128), hr, s, k)
        gsums.append(gsum)

    lw1 = pad48(lW1)
    lb1p = jnp.pad(lb1, (0, DP - 45)).reshape(1, DP)
    lw2 = jnp.pad(lW2, ((0, DP - 45), (0, 32 - 22)))
    lb2p = jnp.pad(lb2, (0, 32 - 22)).reshape(1, 32)
    lw3 = jnp.pad(lW3, ((0, 32 - 22), (0, 8 - 1)))
    lb3p = jnp.pad(lb3, (0, 8 - 1)).reshape(1, 8)
    o = _tc_mlp(gsums[0], gsums[1], gsums[2], lw1, lb1p, lw2, lb2p, lw3, lb3p)
    return o[0, :1]
